# Initial kernel scaffold; baseline (speedup 1.0000x reference)
import jax
import jax.numpy as jnp
from jax.experimental import pallas as pl


def kernel(lookup, table):
    raise NotImplementedError("write your pallas kernel here")


# speedup vs baseline: 1.6847x; 1.6847x over previous
import functools

import jax
import jax.numpy as jnp
from jax import lax
from jax.experimental import pallas as pl
from jax.experimental.pallas import tpu as pltpu
from jax.experimental.pallas import tpu_sc as plsc

D = 64
CH = 128


@functools.lru_cache(maxsize=None)
def _make(NW, NC, NCH, Btot):
    mesh = plsc.VectorSubcoreMesh(core_axis_name="c", subcore_axis_name="s")

    @functools.partial(
        pl.kernel,
        mesh=mesh,
        compiler_params=pltpu.CompilerParams(use_tc_tiling_on_sc=False),
        out_type=jax.ShapeDtypeStruct((Btot, D), jnp.float32),
        scratch_types=[
            pltpu.VMEM((NCH, CH), jnp.int32),
            pltpu.VMEM((CH, D), jnp.float32),
            pltpu.SemaphoreType.DMA,
        ],
    )
    def k(idx_hbm, table_hbm, out_hbm, idx_v, rows_v, gsem):
        wid = lax.axis_index("s") * NC + lax.axis_index("c")
        pltpu.sync_copy(idx_hbm.at[wid], idx_v)

        def body(c, carry):
            pltpu.async_copy(table_hbm.at[idx_v.at[c]], rows_v, gsem).wait()
            off = (wid * NCH + c) * CH
            pltpu.sync_copy(rows_v, out_hbm.at[pl.ds(off, CH)])
            return carry

        lax.fori_loop(0, NCH, body, 0)

    return k


def kernel(lookup, table):
    B, H = lookup.shape
    info = plsc.get_sparse_core_info()
    NC, NS = info.num_cores, info.num_subcores
    NW = NC * NS
    Btot = B * H
    assert Btot % (NW * CH) == 0
    NCH = Btot // (NW * CH)
    idx = lookup.reshape(NW, NCH, CH)
    out = _make(NW, NC, NCH, Btot)(idx, table)
    return out.reshape(B, H, D)


# speedup vs baseline: 1.8800x; 1.1159x over previous
import functools

import jax
import jax.numpy as jnp
from jax import lax
from jax.experimental import pallas as pl
from jax.experimental.pallas import tpu as pltpu
from jax.experimental.pallas import tpu_sc as plsc

D = 64
CH = 128
NBUF = 4
SLOTS = 2 * NBUF


@functools.lru_cache(maxsize=None)
def _make(NW, NC, NCH, Btot):
    mesh = plsc.VectorSubcoreMesh(core_axis_name="c", subcore_axis_name="s")
    R = NCH
    G = R // SLOTS
    assert R % SLOTS == 0 and G >= 2

    @functools.partial(
        pl.kernel,
        mesh=mesh,
        compiler_params=pltpu.CompilerParams(use_tc_tiling_on_sc=False),
        out_type=jax.ShapeDtypeStruct((Btot, D), jnp.float32),
        scratch_types=[
            pltpu.VMEM((NCH, CH), jnp.int32),
            pltpu.VMEM((SLOTS, CH, D), jnp.float32),
            pltpu.SemaphoreType.DMA,
            pltpu.SemaphoreType.DMA,
        ],
    )
    def k(idx_hbm, table_hbm, out_hbm, idx_v, rows_v, gsem, ssem):
        wid = lax.axis_index("s") * NC + lax.axis_index("c")
        pltpu.sync_copy(idx_hbm.at[wid], idx_v)
        obase = wid * NCH

        def fire_g(c, slot):
            pltpu.async_copy(table_hbm.at[idx_v.at[c]], rows_v.at[slot], gsem)

        def wait_g(slot):
            pltpu.make_async_copy(
                table_hbm.at[idx_v.at[0]], rows_v.at[slot], gsem).wait()

        def fire_s(c, slot):
            pltpu.async_copy(
                rows_v.at[slot], out_hbm.at[pl.ds((obase + c) * CH, CH)], ssem)

        def wait_s():
            pltpu.make_async_copy(
                rows_v.at[0], out_hbm.at[pl.ds(obase * CH, CH)], ssem).wait()

        for b in range(NBUF):
            fire_g(b, b)

        for b in range(SLOTS):
            wait_g(b)
            fire_s(b, b)
            if b >= NBUF:
                wait_s()
            fire_g(b + NBUF, (b + NBUF) % SLOTS)

        def group(g, carry):
            c0 = g * SLOTS
            for b in range(SLOTS):
                wait_g(b)
                fire_s(c0 + b, b)
                wait_s()
                fire_g(c0 + b + NBUF, (b + NBUF) % SLOTS)
            return carry

        lax.fori_loop(1, G - 1, group, 0)

        c0 = (G - 1) * SLOTS
        for b in range(SLOTS):
            wait_g(b)
            fire_s(c0 + b, b)
            if b < NBUF:
                wait_s()
                fire_g(c0 + b + NBUF, (b + NBUF) % SLOTS)

        for _ in range(SLOTS):
            wait_s()

    return k


def kernel(lookup, table):
    B, H = lookup.shape
    info = plsc.get_sparse_core_info()
    NC, NS = info.num_cores, info.num_subcores
    NW = NC * NS
    Btot = B * H
    assert Btot % (NW * CH) == 0
    NCH = Btot // (NW * CH)
    idx = lookup.reshape(NW, NCH, CH)
    out = _make(NW, NC, NCH, Btot)(idx, table)
    return out.reshape(B, H, D)
